# baseline (speedup 1.0000x reference)
import jax
import jax.numpy as jnp
from jax.experimental import pallas as pl


BLK = 256


def _add_kernel(x_ref, emb_ref, out_ref):
    out_ref[...] = x_ref[...] + emb_ref[...][None, :, :]


def kernel(x, emb):
    batch, seq_len, d_model = x.shape
    grid = (seq_len // BLK, batch)
    return pl.pallas_call(
        _add_kernel,
        grid=grid,
        in_specs=[
            pl.BlockSpec((1, BLK, d_model), lambda i, j: (j, i, 0)),
            pl.BlockSpec((BLK, d_model), lambda i, j: (i, 0)),
        ],
        out_specs=pl.BlockSpec((1, BLK, d_model), lambda i, j: (j, i, 0)),
        out_shape=jax.ShapeDtypeStruct((batch, seq_len, d_model), x.dtype),
    )(x, emb)


# speedup vs baseline: 1.1381x; 1.1381x over previous
import jax
import jax.numpy as jnp
from jax.experimental import pallas as pl


BLK = 128


def _add_kernel(x_ref, emb_ref, out_ref):
    out_ref[...] = x_ref[...] + emb_ref[...][None, :, :]


def kernel(x, emb):
    batch, seq_len, d_model = x.shape
    grid = (seq_len // BLK,)
    return pl.pallas_call(
        _add_kernel,
        grid=grid,
        in_specs=[
            pl.BlockSpec((batch, BLK, d_model), lambda i: (0, i, 0)),
            pl.BlockSpec((BLK, d_model), lambda i: (i, 0)),
        ],
        out_specs=pl.BlockSpec((batch, BLK, d_model), lambda i: (0, i, 0)),
        out_shape=jax.ShapeDtypeStruct((batch, seq_len, d_model), x.dtype),
    )(x, emb)
